# baseline (speedup 1.0000x reference)
import functools

import jax
import jax.numpy as jnp
from jax import lax
from jax.experimental import pallas as pl
from jax.experimental.pallas import tpu as pltpu
from jax.experimental.pallas import tpu_sc as plsc

VOCAB = 1000
VOC_PAD = 1024
N_EMBD = 32
B, T = 1024, 200
BT = B * T

NC, NS = 2, 16
BH = B // NC
NBG = BH // 128
NV = 64
NV_LAST = VOCAB - 15 * NV


def _table_body(w_ref, emb_ref, b_ref, out_ref):
    ft = lax.dot_general(
        w_ref[...], emb_ref[...],
        (((0,), (1,)), ((), ())),
        preferred_element_type=jnp.float32,
    ) + b_ref[...]
    for i in range(VOC_PAD // 128):
        out_ref[:, i, :] = ft[:, i * 128:(i + 1) * 128]


def _fused_table_t(emb_table, W, b):
    emb_pad = jnp.pad(emb_table, ((0, VOC_PAD - VOCAB), (0, 0)))
    return pl.pallas_call(
        _table_body,
        out_shape=jax.ShapeDtypeStruct((VOCAB, VOC_PAD // 128, 128),
                                       jnp.float32),
    )(W, emb_pad, b.reshape(VOCAB, 1))


_mesh = plsc.VectorSubcoreMesh(core_axis_name="c", subcore_axis_name="s")


@functools.partial(
    pl.kernel,
    mesh=_mesh,
    out_type=jax.ShapeDtypeStruct((T, VOCAB, B), jnp.float32),
    scratch_types=[
        pltpu.VMEM((NV, VOC_PAD // 128, 128), jnp.float32),
        pltpu.VMEM((BH,), jnp.int32),
        pltpu.VMEM((BH,), jnp.int32),
        pltpu.VMEM((NV, 128), jnp.float32),
        pltpu.VMEM((NV, 128), jnp.float32),
        pltpu.SemaphoreType.DMA,
        pltpu.SemaphoreType.DMA,
        pltpu.SemaphoreType.DMA,
        pltpu.SemaphoreType.DMA,
    ],
    compiler_params=pltpu.CompilerParams(
        use_tc_tiling_on_sc=True,
        needs_layout_passes=False,
        disable_bounds_checks=True,
    ),
)
def _sc_gather(ft_hbm, idxt_hbm, out_hbm, tbl, ib0, ib1, ob0, ob1,
               isem0, isem1, osem0, osem1):
    c = lax.axis_index("c")
    s = lax.axis_index("s")
    v0 = s * NV
    b_base = c * BH
    last = s == NS - 1
    nv = jnp.where(last, NV_LAST, NV)

    @pl.when(jnp.logical_not(last))
    def _():
        pltpu.sync_copy(ft_hbm.at[pl.ds(v0, NV)], tbl)

    @pl.when(last)
    def _():
        pltpu.sync_copy(
            ft_hbm.at[pl.ds(15 * NV, NV_LAST)], tbl.at[pl.ds(0, NV_LAST)]
        )

    def fetch_idx(t, ib, sem):
        pltpu.async_copy(idxt_hbm.at[t, pl.ds(b_base, BH)], ib, sem)

    def wait_idx(ib, sem):
        pltpu.make_async_copy(
            idxt_hbm.at[0, pl.ds(b_base, BH)], ib, sem
        ).wait()

    def start_out(t, bg, ob, sem):
        b0 = b_base + bg * 128

        @pl.when(jnp.logical_not(last))
        def _():
            pltpu.async_copy(
                ob, out_hbm.at[t, pl.ds(v0, NV), pl.ds(b0, 128)], sem
            )

        @pl.when(last)
        def _():
            pltpu.async_copy(
                ob.at[pl.ds(0, NV_LAST)],
                out_hbm.at[t, pl.ds(15 * NV, NV_LAST), pl.ds(b0, 128)],
                sem,
            )

    def wait_out(ob, sem):
        @pl.when(jnp.logical_not(last))
        def _():
            pltpu.make_async_copy(
                ob, out_hbm.at[0, pl.ds(0, NV), pl.ds(0, 128)], sem
            ).wait()

        @pl.when(last)
        def _():
            pltpu.make_async_copy(
                ob.at[pl.ds(0, NV_LAST)],
                out_hbm.at[0, pl.ds(0, NV_LAST), pl.ds(0, 128)],
                sem,
            ).wait()

    def compute_bg(t, bg, ib, ob):
        his = []
        los = []
        for l in range(8):
            idx16 = ib[pl.ds(bg * 128 + l * 16, 16)]
            his.append(lax.shift_right_logical(idx16, 7))
            los.append(lax.bitwise_and(idx16, 127))

        def vblock(vb, carry):
            v8 = vb * 8
            for k in range(8):
                vv = jnp.full((16,), v8 + k, jnp.int32)
                for l in range(8):
                    val = plsc.load_gather(tbl, [vv, his[l], los[l]])
                    ob[v8 + k, pl.ds(l * 16, 16)] = val
            return carry

        lax.fori_loop(0, NV // 8, vblock, 0)

    def handle_t(t, ib, isem, obufs, osems):
        wait_idx(ib, isem)
        for bg in range(NBG):
            ob = obufs[bg % 2]
            sem = osems[bg % 2]
            wait_out(ob, sem)
            compute_bg(t, bg, ib, ob)
            start_out(t, bg, ob, sem)

    start_out(0, 0, ob0, osem0)
    start_out(0, 1, ob1, osem1)

    fetch_idx(0, ib0, isem0)
    fetch_idx(1, ib1, isem1)

    def body(p, carry):
        t0 = 2 * p
        handle_t(t0, ib0, isem0, (ob0, ob1), (osem0, osem1))
        fetch_idx(jnp.minimum(t0 + 2, T - 2), ib0, isem0)
        handle_t(t0 + 1, ib1, isem1, (ob0, ob1), (osem0, osem1))
        fetch_idx(jnp.minimum(t0 + 3, T - 1), ib1, isem1)
        return carry

    lax.fori_loop(0, T // 2, body, 0)
    wait_idx(ib0, isem0)
    wait_idx(ib1, isem1)
    wait_out(ob0, osem0)
    wait_out(ob1, osem1)


def kernel(idx, emb_table, W, b):
    ft = _fused_table_t(emb_table, W, b)
    idxt = idx.T.astype(jnp.int32)
    out = _sc_gather(ft, idxt)
    return out.transpose(2, 0, 1)


# speedup vs baseline: 1.5137x; 1.5137x over previous
import functools

import jax
import jax.numpy as jnp
from jax import lax
from jax.experimental import pallas as pl
from jax.experimental.pallas import tpu as pltpu
from jax.experimental.pallas import tpu_sc as plsc
VOCAB = 1000
WMAIN = 896
WTAIL = VOCAB - WMAIN
N_EMBD = 32
B, T = 1024, 200
BT = B * T

NC, NS = 2, 16
NW = NC * NS
B_PER_W = BT // NW
CH = 32
N_CH = B_PER_W // CH
NP = N_CH // 2

_TAIL_OFFS = (0, 16, 32, 48, 64, 80)


def _table_body(emb_ref, wa_ref, wb_ref, ba_ref, bb_ref, outa_ref, outb_ref):
    e = emb_ref[...]
    outa_ref[...] = (
        jnp.dot(e, wa_ref[...], preferred_element_type=jnp.float32)
        + ba_ref[...]
    )
    outb_ref[...] = (
        jnp.dot(e, wb_ref[...], preferred_element_type=jnp.float32)
        + bb_ref[...]
    )


def _fused_tables(emb_table, W, b):
    wa = W[:, :WMAIN]
    wb = jnp.pad(W[:, WMAIN:], ((0, 0), (0, 128 - WTAIL)))
    ba = b[:WMAIN].reshape(1, WMAIN)
    bb = jnp.pad(b[WMAIN:], (0, 128 - WTAIL)).reshape(1, 128)
    return pl.pallas_call(
        _table_body,
        out_shape=(
            jax.ShapeDtypeStruct((VOCAB, WMAIN), jnp.float32),
            jax.ShapeDtypeStruct((VOCAB, 128), jnp.float32),
        ),
    )(emb_table, wa, wb, ba, bb)


_mesh = plsc.VectorSubcoreMesh(core_axis_name="c", subcore_axis_name="s")


@functools.partial(
    pl.kernel,
    mesh=_mesh,
    out_type=jax.ShapeDtypeStruct((BT, VOCAB), jnp.float32),
    scratch_types=[
        pltpu.VMEM((B_PER_W,), jnp.int32),
        pltpu.VMEM((CH, VOCAB), jnp.float32),
        pltpu.VMEM((CH, VOCAB), jnp.float32),
        pltpu.VMEM((CH, 128), jnp.float32),
        pltpu.VMEM((CH, 128), jnp.float32),
        pltpu.SemaphoreType.DMA,
        pltpu.SemaphoreType.DMA,
        pltpu.SemaphoreType.DMA,
        pltpu.SemaphoreType.DMA,
    ],
    compiler_params=pltpu.CompilerParams(
        use_tc_tiling_on_sc=True, needs_layout_passes=False
    ),
)
def _sc_gather(tbla_hbm, tblb_hbm, idx_hbm, out_hbm, idx_v, rows_a, rows_b,
               tail_a, tail_b, gsem_a, gsem_b, ssem_a, ssem_b):
    c = lax.axis_index("c")
    s = lax.axis_index("s")
    wid = s * NC + c
    base = wid * B_PER_W

    pltpu.sync_copy(idx_hbm.at[pl.ds(base, B_PER_W)], idx_v)

    def start_gather(g, rows, tail, sem):
        idxs = idx_v.at[pl.ds(g * CH, CH)]
        pltpu.async_copy(tbla_hbm.at[idxs], rows.at[:, pl.ds(0, WMAIN)], sem)
        pltpu.async_copy(tblb_hbm.at[idxs], tail, sem)

    def wait_gather(rows, tail, sem):
        idxs = idx_v.at[pl.ds(0, CH)]
        pltpu.make_async_copy(
            tbla_hbm.at[idxs], rows.at[:, pl.ds(0, WMAIN)], sem
        ).wait()
        pltpu.make_async_copy(tblb_hbm.at[idxs], tail, sem).wait()

    def fix_tail(rows, tail):
        lanes = lax.iota(jnp.int32, 16)
        rem_mask = lanes < (WTAIL - 96)
        for r in range(CH):
            for off in _TAIL_OFFS:
                rows[r, pl.ds(WMAIN + off, 16)] = tail[r, pl.ds(off, 16)]
            x = tail[r, pl.ds(96, 16)]
            plsc.store_scatter(
                rows,
                [jnp.full((16,), r, jnp.int32), WMAIN + 96 + lanes],
                x,
                mask=rem_mask,
            )

    def start_scatter(g, rows, sem):
        pltpu.async_copy(rows, out_hbm.at[pl.ds(base + g * CH, CH)], sem)

    def wait_scatter(rows, sem):
        pltpu.make_async_copy(
            rows, out_hbm.at[pl.ds(base, CH)], sem
        ).wait()

    start_gather(0, rows_a, tail_a, gsem_a)
    start_gather(1, rows_b, tail_b, gsem_b)

    def body(p, carry):
        g0 = 2 * p
        wait_gather(rows_a, tail_a, gsem_a)
        fix_tail(rows_a, tail_a)
        start_scatter(g0, rows_a, ssem_a)
        wait_gather(rows_b, tail_b, gsem_b)
        fix_tail(rows_b, tail_b)
        start_scatter(g0 + 1, rows_b, ssem_b)
        wait_scatter(rows_a, ssem_a)
        start_gather(jnp.minimum(g0 + 2, N_CH - 2), rows_a, tail_a, gsem_a)
        wait_scatter(rows_b, ssem_b)
        start_gather(jnp.minimum(g0 + 3, N_CH - 1), rows_b, tail_b, gsem_b)
        return carry

    lax.fori_loop(0, NP, body, 0)
    wait_gather(rows_a, tail_a, gsem_a)
    wait_gather(rows_b, tail_b, gsem_b)


def kernel(idx, emb_table, W, b):
    tbla, tblb = _fused_tables(emb_table, W, b)
    flat_idx = idx.reshape(-1).astype(jnp.int32)
    out = _sc_gather(tbla, tblb, flat_idx)
    return out.reshape(B, T, VOCAB)
